# baseline (speedup 1.0000x reference)
import functools

import jax
import jax.numpy as jnp
from jax import lax
from jax.experimental import pallas as pl
from jax.experimental.pallas import tpu as pltpu
from jax.experimental.pallas import tpu_sc as plsc

_VOCAB = 1_000_000
_DIM = 32
_BATCH = 16384 * 26

_info = plsc.get_sparse_core_info()
_NC = _info.num_cores
_NS = _info.num_subcores
_NW = _NC * _NS
_B_PER_W = _BATCH // _NW
_NCHUNK = 32
_C = _B_PER_W // _NCHUNK
_CV = _C // 16

_mesh = plsc.VectorSubcoreMesh(core_axis_name="c", subcore_axis_name="s")


@functools.partial(
    pl.kernel,
    mesh=_mesh,
    out_type=jax.ShapeDtypeStruct((_BATCH // 4, 128), jnp.float32),
    scratch_types=[
        pltpu.VMEM((_B_PER_W,), jnp.int32),
        pltpu.VMEM((_C,), jnp.int32),
        pltpu.VMEM((_C, 128), jnp.float32),
        pltpu.VMEM((_C // 4, 128), jnp.float32),
        pltpu.SemaphoreType.DMA,
        pltpu.SemaphoreType.DMA,
    ],
    compiler_params=pltpu.CompilerParams(needs_layout_passes=False),
)
def _gather_kernel(weight_hbm, idx_hbm, out_hbm, idx_v, gidx, gbuf, ebuf,
                   gsem, ssem):
    sid = lax.axis_index("s")
    wid = sid * _NC + lax.axis_index("c")
    base = wid * _B_PER_W

    pltpu.sync_copy(idx_hbm.at[pl.ds(pl.multiple_of(base, 128), _B_PER_W)],
                    idx_v)

    lanes = lax.iota(jnp.int32, 16)

    def chunk_body(c, _):
        def build(i, _):
            v = idx_v[pl.ds(c * _C + i * 16, 16)]
            gidx[pl.ds(i * 16, 16)] = lax.shift_right_logical(v, 2)
            return 0

        lax.fori_loop(0, _CV, build, 0)
        pltpu.async_copy(weight_hbm.at[gidx], gbuf, gsem).wait()

        def extract(i, _):
            rows = i * 16 + lanes
            v = idx_v[pl.ds(c * _C + i * 16, 16)]
            s = (v & 3) * 32
            er = lax.shift_right_logical(rows, 2)
            ec = (rows & 3) * 32
            for d in range(32):
                vals = plsc.load_gather(gbuf, [rows, s + d])
                plsc.store_scatter(ebuf, [er, ec + d], vals)
            return 0

        lax.fori_loop(0, _CV, extract, 0)
        pltpu.async_copy(
            ebuf,
            out_hbm.at[pl.ds(pl.multiple_of((base + c * _C) // 4, 8),
                             _C // 4)], ssem).wait()
        return 0

    lax.fori_loop(0, _NCHUNK, chunk_body, 0)


def kernel(x, weight):
    w4 = weight.reshape(_VOCAB // 4, 128)
    flat_idx = x.reshape(-1).astype(jnp.int32)
    out4 = _gather_kernel(w4, flat_idx)
    return out4.reshape(x.shape[0], x.shape[1], _DIM)


# speedup vs baseline: 1.0861x; 1.0861x over previous
import functools

import jax
import jax.numpy as jnp
from jax import lax
from jax.experimental import pallas as pl
from jax.experimental.pallas import tpu as pltpu
from jax.experimental.pallas import tpu_sc as plsc

_VOCAB = 1_000_000
_DIM = 32
_BATCH = 16384 * 26

_info = plsc.get_sparse_core_info()
_NC = _info.num_cores
_NS = _info.num_subcores
_NW = _NC * _NS
_B_PER_W = _BATCH // _NW
_C = 256
_NCHUNK = _B_PER_W // _C
_NPAIR = _NCHUNK // 2
_CV = _C // 16

_mesh = plsc.VectorSubcoreMesh(core_axis_name="c", subcore_axis_name="s")


@functools.partial(
    pl.kernel,
    mesh=_mesh,
    out_type=jax.ShapeDtypeStruct((_BATCH // 4, 128), jnp.float32),
    scratch_types=[
        pltpu.VMEM((_B_PER_W,), jnp.int32),
        pltpu.VMEM((_C,), jnp.int32),
        pltpu.VMEM((_C,), jnp.int32),
        pltpu.VMEM((_C, 128), jnp.float32),
        pltpu.VMEM((_C, 128), jnp.float32),
        pltpu.VMEM((_C // 4, 128), jnp.float32),
        pltpu.VMEM((_C // 4, 128), jnp.float32),
        pltpu.SemaphoreType.DMA,
        pltpu.SemaphoreType.DMA,
        pltpu.SemaphoreType.DMA,
        pltpu.SemaphoreType.DMA,
    ],
    compiler_params=pltpu.CompilerParams(needs_layout_passes=False),
)
def _gather_kernel(weight_hbm, idx_hbm, out_hbm, idx_v, gidx0, gidx1,
                   gbuf0, gbuf1, ebuf0, ebuf1, gsem0, gsem1, ssem0, ssem1):
    wid = lax.axis_index("s") * _NC + lax.axis_index("c")
    base = wid * _B_PER_W

    pltpu.sync_copy(idx_hbm.at[pl.ds(pl.multiple_of(base, 128), _B_PER_W)],
                    idx_v)

    lanes = lax.iota(jnp.int32, 16)

    def build(c, gidx):
        def body(i, _):
            v = idx_v[pl.ds(c * _C + i * 16, 16)]
            gidx[pl.ds(i * 16, 16)] = lax.shift_right_logical(v, 2)
            return 0
        lax.fori_loop(0, _CV, body, 0)

    def extract(c, gbuf, ebuf):
        def body(i, _):
            rows = i * 16 + lanes
            v = idx_v[pl.ds(c * _C + i * 16, 16)]
            s = (v & 3) * 32
            er = lax.shift_right_logical(rows, 2)
            ec = (rows & 3) * 32
            for d in range(32):
                vals = plsc.load_gather(gbuf, [rows, s + d])
                plsc.store_scatter(ebuf, [er, ec + d], vals)
            return 0
        lax.fori_loop(0, _CV, body, 0)

    def issue_gather(gidx, gbuf, sem):
        pltpu.async_copy(weight_hbm.at[gidx], gbuf, sem)

    def drain_gather(gbuf, sem):
        pltpu.make_async_copy(weight_hbm.at[pl.ds(0, _C)], gbuf, sem).wait()

    def issue_store(c, ebuf, sem):
        pltpu.async_copy(
            ebuf,
            out_hbm.at[pl.ds(pl.multiple_of((base + c * _C) // 4, 8),
                             _C // 4)], sem)

    def drain_store(ebuf, sem):
        pltpu.make_async_copy(out_hbm.at[pl.ds(0, _C // 4)], ebuf, sem).wait()

    build(0, gidx0)
    issue_gather(gidx0, gbuf0, gsem0)

    def pair_body(i, _):
        a = 2 * i
        b = a + 1
        build(b, gidx1)
        issue_gather(gidx1, gbuf1, gsem1)

        drain_gather(gbuf0, gsem0)

        @pl.when(i > 0)
        def _():
            drain_store(ebuf0, ssem0)
        extract(a, gbuf0, ebuf0)
        issue_store(a, ebuf0, ssem0)

        @pl.when(i < _NPAIR - 1)
        def _():
            build(a + 2, gidx0)
            issue_gather(gidx0, gbuf0, gsem0)

        drain_gather(gbuf1, gsem1)

        @pl.when(i > 0)
        def _():
            drain_store(ebuf1, ssem1)
        extract(b, gbuf1, ebuf1)
        issue_store(b, ebuf1, ssem1)
        return 0

    lax.fori_loop(0, _NPAIR, pair_body, 0)
    drain_store(ebuf0, ssem0)
    drain_store(ebuf1, ssem1)


def kernel(x, weight):
    w4 = weight.reshape(_VOCAB // 4, 128)
    flat_idx = x.reshape(-1).astype(jnp.int32)
    out4 = _gather_kernel(w4, flat_idx)
    return out4.reshape(x.shape[0], x.shape[1], _DIM)


# speedup vs baseline: 1.7234x; 1.5868x over previous
import functools

import jax
import jax.numpy as jnp
from jax import lax
from jax.experimental import pallas as pl
from jax.experimental.pallas import tpu as pltpu
from jax.experimental.pallas import tpu_sc as plsc

_VOCAB = 1_000_000
_DIM = 32
_BATCH = 16384 * 26

_info = plsc.get_sparse_core_info()
_NC = _info.num_cores
_NS = _info.num_subcores
_NW = _NC * _NS
_B_PER_W = _BATCH // _NW
_NCHUNK = 16
_CHUNK = _B_PER_W // _NCHUNK
_NBUF = 4

_mesh = plsc.VectorSubcoreMesh(core_axis_name="c", subcore_axis_name="s")


@functools.partial(
    pl.kernel,
    mesh=_mesh,
    out_type=jax.ShapeDtypeStruct((_BATCH, _DIM), jnp.float32),
    scratch_types=[
        pltpu.VMEM((_NCHUNK, _CHUNK), jnp.int32),
        pltpu.VMEM((_NBUF, _CHUNK, _DIM), jnp.float32),
        pltpu.SemaphoreType.DMA((_NBUF,)),
        pltpu.SemaphoreType.DMA((_NBUF,)),
    ],
    compiler_params=pltpu.CompilerParams(use_tc_tiling_on_sc=False),
)
def _gather_kernel(weight_hbm, idx_hbm, out_hbm, idx_v, rows_v, gsem, ssem):
    wid = lax.axis_index("s") * _NC + lax.axis_index("c")
    base = wid * _B_PER_W

    pltpu.sync_copy(idx_hbm.at[wid], idx_v)

    def gather(c, b):
        return pltpu.async_copy(
            weight_hbm.at[idx_v.at[c]], rows_v.at[b], gsem.at[b])

    def store(c, b):
        return pltpu.async_copy(
            rows_v.at[b], out_hbm.at[pl.ds(base + c * _CHUNK, _CHUNK)],
            ssem.at[b])

    gathers = [gather(b, b) for b in range(_NBUF)]

    stores = [None] * _NBUF
    for c in range(_NCHUNK):
        b = c % _NBUF
        gathers[b].wait()
        stores[b] = store(c, b)
        if c + _NBUF < _NCHUNK:
            stores[b].wait()
            gathers[b] = gather(c + _NBUF, b)
    for b in range(_NBUF):
        if stores[b] is not None:
            stores[b].wait()


def kernel(x, weight):
    flat_idx = x.reshape(_NW, _NCHUNK, _CHUNK).astype(jnp.int32)
    out = _gather_kernel(weight, flat_idx)
    return out.reshape(x.shape[0], x.shape[1], _DIM)


# speedup vs baseline: 1.7242x; 1.0005x over previous
import functools

import jax
import jax.numpy as jnp
from jax import lax
from jax.experimental import pallas as pl
from jax.experimental.pallas import tpu as pltpu
from jax.experimental.pallas import tpu_sc as plsc

_VOCAB = 1_000_000
_DIM = 32
_BATCH = 16384 * 26

_info = plsc.get_sparse_core_info()
_NC = _info.num_cores
_NS = _info.num_subcores
_NW = _NC * _NS
_B_PER_W = _BATCH // _NW
_NCHUNK = 8
_CHUNK = _B_PER_W // _NCHUNK
_NBUF = 2

_mesh = plsc.VectorSubcoreMesh(core_axis_name="c", subcore_axis_name="s")


@functools.partial(
    pl.kernel,
    mesh=_mesh,
    out_type=jax.ShapeDtypeStruct((_BATCH, _DIM), jnp.float32),
    scratch_types=[
        pltpu.VMEM((_NCHUNK, _CHUNK), jnp.int32),
        pltpu.VMEM((_NBUF, _CHUNK, _DIM), jnp.float32),
        pltpu.SemaphoreType.DMA((_NBUF,)),
        pltpu.SemaphoreType.DMA((_NBUF,)),
    ],
    compiler_params=pltpu.CompilerParams(use_tc_tiling_on_sc=False),
)
def _gather_kernel(weight_hbm, idx_hbm, out_hbm, idx_v, rows_v, gsem, ssem):
    wid = lax.axis_index("s") * _NC + lax.axis_index("c")
    base = wid * _B_PER_W

    pltpu.sync_copy(idx_hbm.at[wid], idx_v)

    def gather(c, b):
        return pltpu.async_copy(
            weight_hbm.at[idx_v.at[c]], rows_v.at[b], gsem.at[b])

    def store(c, b):
        return pltpu.async_copy(
            rows_v.at[b], out_hbm.at[pl.ds(base + c * _CHUNK, _CHUNK)],
            ssem.at[b])

    gathers = [gather(b, b) for b in range(_NBUF)]

    stores = [None] * _NBUF
    for c in range(_NCHUNK):
        b = c % _NBUF
        gathers[b].wait()
        stores[b] = store(c, b)
        if c + _NBUF < _NCHUNK:
            stores[b].wait()
            gathers[b] = gather(c + _NBUF, b)
    for b in range(_NBUF):
        if stores[b] is not None:
            stores[b].wait()


def kernel(x, weight):
    flat_idx = x.reshape(_NW, _NCHUNK, _CHUNK).astype(jnp.int32)
    out = _gather_kernel(weight, flat_idx)
    return out.reshape(x.shape[0], x.shape[1], _DIM)
